# baseline (speedup 1.0000x reference)
import functools

import jax
import jax.numpy as jnp
from jax import lax
from jax.experimental import pallas as pl
from jax.experimental.pallas import tpu as pltpu
from jax.experimental.pallas import tpu_sc as plsc

N_NODES = 10000
FEAT = 128
HEADS = 8
CDIM = 16
HFEAT = FEAT // 2
NPAD = 10112
B = 128
NW = 32
ROWS_PER_TILE = NPAD // 16


def _matmul2(x, W_l, W_r):
    def body(x_ref, wl_ref, wr_ref, xl_ref, xr_ref):
        xb = x_ref[...]
        xl_ref[...] = jnp.dot(xb, wl_ref[...], preferred_element_type=jnp.float32)
        xr_ref[...] = jnp.dot(xb, wr_ref[...], preferred_element_type=jnp.float32)

    return pl.pallas_call(
        body,
        grid=(10,),
        in_specs=[
            pl.BlockSpec((N_NODES // 10, FEAT), lambda i: (i, 0)),
            pl.BlockSpec((FEAT, HEADS * CDIM), lambda i: (0, 0)),
            pl.BlockSpec((FEAT, HEADS * CDIM), lambda i: (0, 0)),
        ],
        out_specs=[
            pl.BlockSpec((N_NODES // 10, HEADS * CDIM), lambda i: (i, 0)),
            pl.BlockSpec((N_NODES // 10, HEADS * CDIM), lambda i: (i, 0)),
        ],
        out_shape=[jax.ShapeDtypeStruct((N_NODES, HEADS * CDIM), jnp.float32)] * 2,
    )(x, W_l, W_r)


def _sc_edge_pass(xl0, xl1, xr0, xr1, src_p, dst_p, ea_p, we_flat, att_flat,
                  chunks_per_worker):
    mesh = plsc.VectorSubcoreMesh(core_axis_name="c", subcore_axis_name="s")

    @functools.partial(
        pl.kernel,
        out_type=[
            jax.ShapeDtypeStruct((2, 2, NPAD, HFEAT), jnp.float32),
            jax.ShapeDtypeStruct((2, 2, NPAD, 16), jnp.float32),
        ],
        mesh=mesh,
        compiler_params=pltpu.CompilerParams(use_tc_tiling_on_sc=False),
        scratch_types=[
            pltpu.VMEM((B,), jnp.int32),
            pltpu.VMEM((B,), jnp.int32),
            pltpu.VMEM((B, 16), jnp.float32),
            pltpu.VMEM((B, HFEAT), jnp.float32),
            pltpu.VMEM((B, HFEAT), jnp.float32),
            pltpu.VMEM((B, HFEAT), jnp.float32),
            pltpu.VMEM((B, 16), jnp.float32),
            pltpu.VMEM((FEAT,), jnp.float32),
            pltpu.VMEM((FEAT,), jnp.float32),
            pltpu.VMEM_SHARED((NPAD, HFEAT), jnp.float32),
            pltpu.VMEM_SHARED((NPAD, 16), jnp.float32),
            pltpu.SemaphoreType.DMA,
            pltpu.SemaphoreType.DMA,
        ],
    )
    def sc_kernel(xl0_hbm, xl1_hbm, xr0_hbm, xr1_hbm, src_hbm, dst_hbm,
                  ea_hbm, we_hbm, att_hbm, acc_hbm, den_hbm, srcv, dstv, eav,
                  xl_rows, xr_rows, w_rows, d_rows, wconst, aconst, acc_sh,
                  den_sh, sem1, sem2):
        cid = lax.axis_index("c")
        sid = lax.axis_index("s")
        wid = cid * 16 + sid

        pltpu.sync_copy(we_hbm, wconst)
        pltpu.sync_copy(att_hbm, aconst)

        row0 = sid * ROWS_PER_TILE
        nfull = ROWS_PER_TILE // B
        rem = ROWS_PER_TILE % B

        lane = lax.iota(jnp.int32, 16)
        perms = [jnp.bitwise_xor(lane, k) for k in (8, 4, 2, 1)]
        gdn = lax.GatherDimensionNumbers(
            offset_dims=(), collapsed_slice_dims=(0,), start_index_map=(0,))

        def lane_allsum(v):
            for p in perms:
                v = v + lax.gather(
                    v, p[:, None], gdn, (1,),
                    mode=lax.GatherScatterMode.PROMISE_IN_BOUNDS)
            return v

        for half in range(2):
            xl_h = (xl0_hbm, xl1_hbm)[half]
            xr_h = (xr0_hbm, xr1_hbm)[half]

            @pl.loop(0, B)
            def _(r):
                for k in range(HFEAT // 16):
                    w_rows[r, pl.ds(k * 16, 16)] = jnp.zeros((16,), jnp.float32)
                d_rows[r] = jnp.zeros((16,), jnp.float32)

            for j in range(nfull):
                pltpu.sync_copy(w_rows, acc_sh.at[pl.ds(row0 + j * B, B)])
                pltpu.sync_copy(d_rows, den_sh.at[pl.ds(row0 + j * B, B)])
            if rem:
                pltpu.sync_copy(
                    w_rows.at[pl.ds(0, rem)],
                    acc_sh.at[pl.ds(row0 + nfull * B, rem)],
                )
                pltpu.sync_copy(
                    d_rows.at[pl.ds(0, rem)],
                    den_sh.at[pl.ds(row0 + nfull * B, rem)],
                )
            plsc.subcore_barrier()

            @pl.loop(0, chunks_per_worker)
            def _(t):
                base = (wid * chunks_per_worker + t) * B
                pltpu.sync_copy(src_hbm.at[pl.ds(base, B)], srcv)
                pltpu.sync_copy(dst_hbm.at[pl.ds(base, B)], dstv)
                pltpu.sync_copy(ea_hbm.at[pl.ds(base, B)], eav)
                g1 = pltpu.async_copy(xl_h.at[srcv], xl_rows, sem1)
                g2 = pltpu.async_copy(xr_h.at[dstv], xr_rows, sem2)
                g1.wait()
                g2.wait()

                @plsc.parallel_loop(0, B, unroll=8)
                def _(e):
                    eb = eav[e]
                    den = jnp.zeros((16,), jnp.float32)
                    for h in range(HEADS // 2):
                        sl = pl.ds(h * 16, 16)
                        cl = pl.ds(half * HFEAT + h * 16, 16)
                        xlv = xl_rows[e, sl]
                        u = xlv + xr_rows[e, sl] + eb * wconst[cl]
                        u = jnp.maximum(u, u * 0.2)
                        ex = jnp.exp(lane_allsum(u * aconst[cl]))
                        w_rows[e, sl] = ex * xlv
                        den = jnp.where(lane == h, ex, den)
                    d_rows[e] = den

                pltpu.sync_copy(w_rows, acc_sh.at[dstv], add=True)
                pltpu.sync_copy(d_rows, den_sh.at[dstv], add=True)

            plsc.subcore_barrier()
            for j in range(nfull):
                pltpu.sync_copy(
                    acc_sh.at[pl.ds(row0 + j * B, B)],
                    acc_hbm.at[half, cid, pl.ds(row0 + j * B, B)],
                )
                pltpu.sync_copy(
                    den_sh.at[pl.ds(row0 + j * B, B)],
                    den_hbm.at[half, cid, pl.ds(row0 + j * B, B)],
                )
            if rem:
                pltpu.sync_copy(
                    acc_sh.at[pl.ds(row0 + nfull * B, rem)],
                    acc_hbm.at[half, cid, pl.ds(row0 + nfull * B, rem)],
                )
                pltpu.sync_copy(
                    den_sh.at[pl.ds(row0 + nfull * B, rem)],
                    den_hbm.at[half, cid, pl.ds(row0 + nfull * B, rem)],
                )

    return sc_kernel(xl0, xl1, xr0, xr1, src_p, dst_p, ea_p, we_flat, att_flat)


def _normalize(acc, den, sel, bias2d):
    rows = NPAD // 4

    def body(acc_ref, den_ref, sel_ref, b_ref, out_ref):
        w = jnp.concatenate(
            [acc_ref[0, 0] + acc_ref[0, 1], acc_ref[1, 0] + acc_ref[1, 1]],
            axis=1)
        d = jnp.concatenate(
            [den_ref[0, 0] + den_ref[0, 1], den_ref[1, 0] + den_ref[1, 1]],
            axis=1)
        denr = jnp.dot(d, sel_ref[...], preferred_element_type=jnp.float32)
        out_ref[...] = w / (denr + 1e-16) + b_ref[...]

    return pl.pallas_call(
        body,
        grid=(4,),
        in_specs=[
            pl.BlockSpec((2, 2, rows, HFEAT), lambda i: (0, 0, i, 0)),
            pl.BlockSpec((2, 2, rows, 16), lambda i: (0, 0, i, 0)),
            pl.BlockSpec((32, FEAT), lambda i: (0, 0)),
            pl.BlockSpec((1, FEAT), lambda i: (0, 0)),
        ],
        out_specs=pl.BlockSpec((rows, FEAT), lambda i: (i, 0)),
        out_shape=jax.ShapeDtypeStruct((NPAD, FEAT), jnp.float32),
    )(acc, den, sel, bias2d)


def kernel(x, edge_index, edge_attr, W_l, W_r, W_e, att, bias):
    E = edge_index.shape[1]
    n_chunks = -(-E // B)
    n_chunks = -(-n_chunks // NW) * NW
    e_pad = n_chunks * B
    chunks_per_worker = n_chunks // NW

    src = edge_index[0].astype(jnp.int32)
    dst = edge_index[1].astype(jnp.int32)
    ea = edge_attr.reshape(-1).astype(jnp.float32)
    pad = e_pad - E
    src_p = jnp.concatenate([src, jnp.zeros((pad,), jnp.int32)])
    dst_p = jnp.concatenate([dst, jnp.full((pad,), N_NODES, jnp.int32)])
    ea_p = jnp.concatenate([ea, jnp.zeros((pad,), jnp.float32)])
    ea_p = jnp.broadcast_to(ea_p[:, None], (e_pad, 16))

    we_flat = W_e.reshape(FEAT).astype(jnp.float32)
    att_flat = att.reshape(FEAT).astype(jnp.float32)

    xl, xr = _matmul2(x, W_l, W_r)
    acc, den = _sc_edge_pass(
        xl[:, :HFEAT], xl[:, HFEAT:], xr[:, :HFEAT], xr[:, HFEAT:],
        src_p, dst_p, ea_p, we_flat, att_flat, chunks_per_worker)

    blk = jnp.kron(jnp.eye(HEADS, dtype=jnp.float32),
                   jnp.ones((1, CDIM), jnp.float32))
    sel = jnp.zeros((32, FEAT), jnp.float32)
    sel = sel.at[0:4].set(blk[0:4]).at[16:20].set(blk[4:8])
    out = _normalize(acc, den, sel, bias.reshape(1, FEAT).astype(jnp.float32))
    return out[:N_NODES]


# speedup vs baseline: 1.1489x; 1.1489x over previous
import functools

import jax
import jax.numpy as jnp
from jax import lax
from jax.experimental import pallas as pl
from jax.experimental.pallas import tpu as pltpu
from jax.experimental.pallas import tpu_sc as plsc

N_NODES = 10000
FEAT = 128
HEADS = 8
CDIM = 16
HFEAT = FEAT // 2
NPAD = 10112
B = 128
NW = 32
ROWS_PER_TILE = NPAD // 16


def _matmul2(x, W_l, W_r):
    def body(x_ref, wl_ref, wr_ref, xl_ref, xr_ref):
        xb = x_ref[...]
        xl_ref[...] = jnp.dot(xb, wl_ref[...], preferred_element_type=jnp.float32)
        xr_ref[...] = jnp.dot(xb, wr_ref[...], preferred_element_type=jnp.float32)

    return pl.pallas_call(
        body,
        grid=(16,),
        in_specs=[
            pl.BlockSpec((NPAD // 16, FEAT), lambda i: (i, 0)),
            pl.BlockSpec((FEAT, HEADS * CDIM), lambda i: (0, 0)),
            pl.BlockSpec((FEAT, HEADS * CDIM), lambda i: (0, 0)),
        ],
        out_specs=[
            pl.BlockSpec((NPAD // 16, HEADS * CDIM), lambda i: (i, 0)),
            pl.BlockSpec((NPAD // 16, HEADS * CDIM), lambda i: (i, 0)),
        ],
        out_shape=[jax.ShapeDtypeStruct((NPAD, HEADS * CDIM), jnp.float32)] * 2,
    )(x, W_l, W_r)


def _sc_edge_pass(xl0, xl1, xr0, xr1, src_p, dst_p, ea_p, we_flat, att_flat,
                  chunks_per_worker):
    mesh = plsc.VectorSubcoreMesh(core_axis_name="c", subcore_axis_name="s")

    @functools.partial(
        pl.kernel,
        out_type=[
            jax.ShapeDtypeStruct((2, 2, NPAD, HFEAT), jnp.float32),
            jax.ShapeDtypeStruct((2, 2, NPAD, 16), jnp.float32),
        ],
        mesh=mesh,
        compiler_params=pltpu.CompilerParams(use_tc_tiling_on_sc=False),
        scratch_types=(
            [pltpu.VMEM((B,), jnp.int32)] * 6
            + [
                pltpu.VMEM((3, B, 16), jnp.float32),
                pltpu.VMEM((3, B, HFEAT), jnp.float32),
                pltpu.VMEM((3, B, HFEAT), jnp.float32),
                pltpu.VMEM((B, HFEAT), jnp.float32),
                pltpu.VMEM((B, 16), jnp.float32),
                pltpu.VMEM((FEAT,), jnp.float32),
                pltpu.VMEM((FEAT,), jnp.float32),
                pltpu.VMEM_SHARED((NPAD, HFEAT), jnp.float32),
                pltpu.VMEM_SHARED((NPAD, 16), jnp.float32),
            ]
            + [pltpu.SemaphoreType.DMA] * 9
        ),
    )
    def sc_kernel(xl0_hbm, xl1_hbm, xr0_hbm, xr1_hbm, src_hbm, dst_hbm,
                  ea_hbm, we_hbm, att_hbm, acc_hbm, den_hbm, *scr):
        srcv = scr[0:3]
        dstv = scr[3:6]
        (eav, xl_rows, xr_rows, w_rows, d_rows, wconst, aconst,
         acc_sh, den_sh) = scr[6:15]
        semI = scr[15:18]
        semGa = scr[18:21]
        semGb = scr[21:24]

        cid = lax.axis_index("c")
        sid = lax.axis_index("s")
        wid = cid * 16 + sid
        cpw = chunks_per_worker

        pltpu.sync_copy(we_hbm, wconst)
        pltpu.sync_copy(att_hbm, aconst)

        row0 = sid * ROWS_PER_TILE
        nfull = ROWS_PER_TILE // B
        rem = ROWS_PER_TILE % B

        lane = lax.iota(jnp.int32, 16)
        perms = [jnp.bitwise_xor(lane, k) for k in (8, 4, 2, 1)]
        gdn = lax.GatherDimensionNumbers(
            offset_dims=(), collapsed_slice_dims=(0,), start_index_map=(0,))

        def lane_allsum(v):
            for p in perms:
                v = v + lax.gather(
                    v, p[:, None], gdn, (1,),
                    mode=lax.GatherScatterMode.PROMISE_IN_BOUNDS)
            return v

        def fire_idx(b, c):
            base = (wid * cpw + c) * B
            h1 = pltpu.async_copy(src_hbm.at[pl.ds(base, B)], srcv[b], semI[b])
            h2 = pltpu.async_copy(dst_hbm.at[pl.ds(base, B)], dstv[b], semI[b])
            h3 = pltpu.async_copy(ea_hbm.at[pl.ds(base, B)], eav.at[b], semI[b])
            return h1, h2, h3

        def drain_idx(b):
            pltpu.make_async_copy(
                src_hbm.at[pl.ds(0, B)], srcv[b], semI[b]).wait()
            pltpu.make_async_copy(
                dst_hbm.at[pl.ds(0, B)], dstv[b], semI[b]).wait()
            pltpu.make_async_copy(
                ea_hbm.at[pl.ds(0, B)], eav.at[b], semI[b]).wait()

        for half in range(2):
            xl_h = (xl0_hbm, xl1_hbm)[half]
            xr_h = (xr0_hbm, xr1_hbm)[half]

            def fire_gather(b):
                pltpu.async_copy(xl_h.at[srcv[b]], xl_rows.at[b], semGa[b])
                pltpu.async_copy(xr_h.at[dstv[b]], xr_rows.at[b], semGb[b])

            def drain_gather(b):
                pltpu.make_async_copy(
                    xl_h.at[srcv[b]], xl_rows.at[b], semGa[b]).wait()
                pltpu.make_async_copy(
                    xr_h.at[dstv[b]], xr_rows.at[b], semGb[b]).wait()

            @plsc.parallel_loop(0, B, unroll=4)
            def _(r):
                for k in range(HFEAT // 16):
                    w_rows[r, pl.ds(k * 16, 16)] = jnp.zeros((16,), jnp.float32)
                d_rows[r] = jnp.zeros((16,), jnp.float32)

            for j in range(nfull):
                pltpu.sync_copy(w_rows, acc_sh.at[pl.ds(row0 + j * B, B)])
                pltpu.sync_copy(d_rows, den_sh.at[pl.ds(row0 + j * B, B)])
            if rem:
                pltpu.sync_copy(
                    w_rows.at[pl.ds(0, rem)],
                    acc_sh.at[pl.ds(row0 + nfull * B, rem)],
                )
                pltpu.sync_copy(
                    d_rows.at[pl.ds(0, rem)],
                    den_sh.at[pl.ds(row0 + nfull * B, rem)],
                )
            plsc.subcore_barrier()

            pro = [fire_idx(b, b) for b in range(3)]
            for b in range(2):
                for h in pro[b]:
                    h.wait()
                fire_gather(b)

            @pl.loop(0, cpw, step=3)
            def _(t):
                for b in range(3):
                    c = t + b
                    drain_gather(b)

                    @plsc.parallel_loop(0, B, unroll=4)
                    def _(e):
                        eb = eav[b, e]
                        den = jnp.zeros((16,), jnp.float32)
                        for h in range(HEADS // 2):
                            sl = pl.ds(h * 16, 16)
                            cl = pl.ds(half * HFEAT + h * 16, 16)
                            xlv = xl_rows[b, e, sl]
                            u = xlv + xr_rows[b, e, sl] + eb * wconst[cl]
                            u = jnp.maximum(u, u * 0.2)
                            ex = jnp.exp(lane_allsum(u * aconst[cl]))
                            w_rows[e, sl] = ex * xlv
                            den = jnp.where(lane == h, ex, den)
                        d_rows[e] = den

                    pltpu.sync_copy(w_rows, acc_sh.at[dstv[b]], add=True)
                    pltpu.sync_copy(d_rows, den_sh.at[dstv[b]], add=True)

                    fire_idx(b, c + 3)
                    b2 = (b + 2) % 3
                    drain_idx(b2)
                    fire_gather(b2)

            for b in range(2):
                drain_gather(b)
            drain_idx(2)

            plsc.subcore_barrier()
            for j in range(nfull):
                pltpu.sync_copy(
                    acc_sh.at[pl.ds(row0 + j * B, B)],
                    acc_hbm.at[half, cid, pl.ds(row0 + j * B, B)],
                )
                pltpu.sync_copy(
                    den_sh.at[pl.ds(row0 + j * B, B)],
                    den_hbm.at[half, cid, pl.ds(row0 + j * B, B)],
                )
            if rem:
                pltpu.sync_copy(
                    acc_sh.at[pl.ds(row0 + nfull * B, rem)],
                    acc_hbm.at[half, cid, pl.ds(row0 + nfull * B, rem)],
                )
                pltpu.sync_copy(
                    den_sh.at[pl.ds(row0 + nfull * B, rem)],
                    den_hbm.at[half, cid, pl.ds(row0 + nfull * B, rem)],
                )

    return sc_kernel(xl0, xl1, xr0, xr1, src_p, dst_p, ea_p, we_flat, att_flat)


def _normalize(acc, den, sel, bias2d):
    rows = NPAD // 4

    def body(acc_ref, den_ref, sel_ref, b_ref, out_ref):
        w = jnp.concatenate(
            [acc_ref[0, 0] + acc_ref[0, 1], acc_ref[1, 0] + acc_ref[1, 1]],
            axis=1)
        d = jnp.concatenate(
            [den_ref[0, 0] + den_ref[0, 1], den_ref[1, 0] + den_ref[1, 1]],
            axis=1)
        denr = jnp.dot(d, sel_ref[...], preferred_element_type=jnp.float32)
        out_ref[...] = w / (denr + 1e-16) + b_ref[...]

    return pl.pallas_call(
        body,
        grid=(4,),
        in_specs=[
            pl.BlockSpec((2, 2, rows, HFEAT), lambda i: (0, 0, i, 0)),
            pl.BlockSpec((2, 2, rows, 16), lambda i: (0, 0, i, 0)),
            pl.BlockSpec((32, FEAT), lambda i: (0, 0)),
            pl.BlockSpec((1, FEAT), lambda i: (0, 0)),
        ],
        out_specs=pl.BlockSpec((rows, FEAT), lambda i: (i, 0)),
        out_shape=jax.ShapeDtypeStruct((NPAD, FEAT), jnp.float32),
    )(acc, den, sel, bias2d)


def kernel(x, edge_index, edge_attr, W_l, W_r, W_e, att, bias):
    E = edge_index.shape[1]
    n_chunks = -(-E // B)
    n_chunks = -(-n_chunks // (3 * NW)) * (3 * NW)
    e_pad = n_chunks * B
    chunks_per_worker = n_chunks // NW

    src = edge_index[0].astype(jnp.int32)
    dst = edge_index[1].astype(jnp.int32)
    ea = edge_attr.reshape(-1).astype(jnp.float32)
    pad = e_pad + 3 * B - E
    src_p = jnp.concatenate([src, jnp.zeros((pad,), jnp.int32)])
    dst_p = jnp.concatenate([dst, jnp.full((pad,), N_NODES, jnp.int32)])
    ea_p = jnp.concatenate([ea, jnp.zeros((pad,), jnp.float32)])
    ea_p = jnp.broadcast_to(ea_p[:, None], (e_pad + 3 * B, 16))

    we_flat = W_e.reshape(FEAT).astype(jnp.float32)
    att_flat = att.reshape(FEAT).astype(jnp.float32)

    x_p = jnp.zeros((NPAD, FEAT), x.dtype).at[:N_NODES].set(x)
    xl, xr = _matmul2(x_p, W_l, W_r)
    acc, den = _sc_edge_pass(
        xl[:, :HFEAT], xl[:, HFEAT:], xr[:, :HFEAT], xr[:, HFEAT:],
        src_p, dst_p, ea_p, we_flat, att_flat, chunks_per_worker)

    blk = jnp.kron(jnp.eye(HEADS, dtype=jnp.float32),
                   jnp.ones((1, CDIM), jnp.float32))
    sel = jnp.zeros((32, FEAT), jnp.float32)
    sel = sel.at[0:4].set(blk[0:4]).at[16:20].set(blk[4:8])
    out = _normalize(acc, den, sel, bias.reshape(1, FEAT).astype(jnp.float32))
    return out[:N_NODES]
